# baseline (speedup 1.0000x reference)
import functools

import jax
import jax.numpy as jnp
import numpy as np
from jax import lax
from jax.experimental import pallas as pl
from jax.experimental.pallas import tpu as pltpu
from jax.experimental.pallas import tpu_sc as plsc

_N_FIELDS = 26
_VOCAB_P1 = 100001
_EMBED = 32
_BATCH = 16384

_NC, _NS, _L = 2, 16, 16
_NW = _NC * _NS
_BPW = _BATCH // _NW
_ROWS_PW = _BPW * _N_FIELDS
_GROUP = 128
_NGROUP = _ROWS_PW // _GROUP
_CHUNK_B = 64
_CHUNK_ROWS = _CHUNK_B * _N_FIELDS
_GPC = _CHUNK_ROWS // _GROUP
_NCHUNK = _BPW // _CHUNK_B
_PERIOD = 13


def _body(c_hbm, offs_hbm, tbl_hbm, out_hbm, idx_v, offs_v, rows0, rows1,
          out_v, sem0, sem1):
    wid = lax.axis_index("s") * _NC + lax.axis_index("c")

    pltpu.sync_copy(c_hbm.at[wid], idx_v)
    pltpu.sync_copy(offs_hbm, offs_v)

    @pl.loop(0, _NGROUP // _PERIOD)
    def _offset_add(q):
        g0 = q * _PERIOD
        for r in range(_PERIOD):
            for v in range(_GROUP // _L):
                sl = pl.ds(v * _L, _L)
                idx_v[g0 + r, sl] = idx_v[g0 + r, sl] + offs_v[r, sl]

    def fire(k, buf, sem):
        cps = []
        for g in range(_GPC):
            cps.append(pltpu.async_copy(
                tbl_hbm.at[idx_v.at[k * _GPC + g]],
                buf.at[pl.ds(g * _GROUP, _GROUP)], sem))
        return cps

    inv = jnp.float32(1.0 / _N_FIELDS)

    def accumulate(buf):
        @pl.loop(0, _CHUNK_B)
        def _acc(b):
            base = b * _N_FIELDS
            lo = buf[base, pl.ds(0, _L)]
            hi = buf[base, pl.ds(_L, _L)]
            for f in range(1, _N_FIELDS):
                lo = lo + buf[base + f, pl.ds(0, _L)]
                hi = hi + buf[base + f, pl.ds(_L, _L)]
            out_v[b, pl.ds(0, _L)] = lo * inv
            out_v[b, pl.ds(_L, _L)] = hi * inv

    bufs = (rows0, rows1)
    sems = (sem0, sem1)
    pending = fire(0, bufs[0], sems[0])
    for k in range(_NCHUNK):
        if k + 1 < _NCHUNK:
            nxt = fire(k + 1, bufs[(k + 1) % 2], sems[(k + 1) % 2])
        for cp in pending:
            cp.wait()
        accumulate(bufs[k % 2])
        pltpu.sync_copy(out_v, out_hbm.at[pl.ds(wid * _BPW + k * _CHUNK_B,
                                                _CHUNK_B)])
        if k + 1 < _NCHUNK:
            pending = nxt


@jax.jit
def _sc_embed(c3, offs, tbl):
    mesh = plsc.VectorSubcoreMesh(core_axis_name="c", subcore_axis_name="s",
                                  num_cores=_NC, num_subcores=_NS)
    return pl.kernel(
        _body,
        out_type=jax.ShapeDtypeStruct((_BATCH, _EMBED), jnp.float32),
        mesh=mesh,
        scratch_types=[
            pltpu.VMEM((_NGROUP, _GROUP), jnp.int32),
            pltpu.VMEM((_PERIOD, _GROUP), jnp.int32),
            pltpu.VMEM((_CHUNK_ROWS, _EMBED), jnp.float32),
            pltpu.VMEM((_CHUNK_ROWS, _EMBED), jnp.float32),
            pltpu.VMEM((_CHUNK_B, _EMBED), jnp.float32),
            pltpu.SemaphoreType.DMA,
            pltpu.SemaphoreType.DMA,
        ],
        compiler_params=pltpu.CompilerParams(use_tc_tiling_on_sc=False),
    )(c3, offs, tbl)


def kernel(c, tables):
    c3 = c.reshape(_NW, _NGROUP, _GROUP)
    tbl = tables.reshape(_N_FIELDS * _VOCAB_P1, _EMBED)
    offs = jnp.asarray(
        (np.arange(_PERIOD * _GROUP).reshape(_PERIOD, _GROUP) % _N_FIELDS)
        * _VOCAB_P1, dtype=np.int32)
    return _sc_embed(c3, offs, tbl)


# speedup vs baseline: 11.2055x; 11.2055x over previous
import jax
import jax.numpy as jnp
import numpy as np
from jax import lax
from jax.experimental import pallas as pl
from jax.experimental.pallas import tpu as pltpu
from jax.experimental.pallas import tpu_sc as plsc

_N_FIELDS = 26
_VOCAB_P1 = 100001
_VP = 100096
_EMBED = 32
_BATCH = 16384

_NC, _NS, _L = 2, 16, 16
_NW = _NC * _NS
_BPW = _BATCH // _NW
_ROWS_PW = _BPW * _N_FIELDS
_GROUP = 128
_NGROUP = _ROWS_PW // _GROUP
_CHUNK_B = 64
_CHUNK_ROWS = _CHUNK_B * _N_FIELDS
_GPC = _CHUNK_ROWS // _GROUP
_NCHUNK = _BPW // _CHUNK_B
_PERIOD = 13

_BLKV = 2176
_NVB = _VP // _BLKV
_RB = _BLKV // 8
_R128 = _N_FIELDS * _VP * 16 // 128


def _repack_body(x_ref, o_ref):
    x = x_ref[0]
    lo = lax.bitcast_convert_type(
        x[0:16, :].astype(jnp.bfloat16), jnp.uint16).astype(jnp.uint32)
    hi = lax.bitcast_convert_type(
        x[16:32, :].astype(jnp.bfloat16), jnp.uint16).astype(jnp.uint32)
    packed = lax.bitcast_convert_type(lo | (hi << 16), jnp.int32)
    p3 = packed.T.reshape(_RB, 8, 16)
    for a in range(8):
        o_ref[:, a * 16:(a + 1) * 16] = p3[:, a, :]


@jax.jit
def _repack(t2):
    return pl.pallas_call(
        _repack_body,
        grid=(_N_FIELDS, _NVB),
        in_specs=[pl.BlockSpec((1, _EMBED, _BLKV), lambda f, v: (f, 0, v))],
        out_specs=pl.BlockSpec((_RB, 128), lambda f, v: (f * _NVB + v, 0)),
        out_shape=jax.ShapeDtypeStruct((_R128, 128), jnp.int32),
    )(t2)


def _body(c_hbm, offs_hbm, tbl_hbm, out_hbm, idx_v, offs_v, rows0, rows1,
          out_v, sem0, sem1):
    wid = lax.axis_index("s") * _NC + lax.axis_index("c")

    pltpu.sync_copy(c_hbm.at[wid], idx_v)
    pltpu.sync_copy(offs_hbm, offs_v)

    @pl.loop(0, _NGROUP // _PERIOD)
    def _offset_add(q):
        g0 = q * _PERIOD
        for r in range(_PERIOD):
            for v in range(_GROUP // _L):
                sl = pl.ds(v * _L, _L)
                idx_v[g0 + r, sl] = idx_v[g0 + r, sl] + offs_v[r, sl]

    def fire(k, buf, sem):
        cps = []
        for g in range(_GPC):
            cps.append(pltpu.async_copy(
                tbl_hbm.at[idx_v.at[k * _GPC + g]],
                buf.at[pl.ds(g * _GROUP, _GROUP)], sem))
        return cps

    inv = jnp.float32(1.0 / _N_FIELDS)
    msk = jnp.int32(-65536)

    def accumulate(buf):
        @pl.loop(0, _CHUNK_B)
        def _acc(b):
            base = b * _N_FIELDS
            v = buf[base, :]
            lo = lax.bitcast_convert_type(v << 16, jnp.float32)
            hi = lax.bitcast_convert_type(v & msk, jnp.float32)
            for f in range(1, _N_FIELDS):
                v = buf[base + f, :]
                lo = lo + lax.bitcast_convert_type(v << 16, jnp.float32)
                hi = hi + lax.bitcast_convert_type(v & msk, jnp.float32)
            out_v[b, pl.ds(0, _L)] = lo * inv
            out_v[b, pl.ds(_L, _L)] = hi * inv

    bufs = (rows0, rows1)
    sems = (sem0, sem1)
    pending = fire(0, bufs[0], sems[0])
    for k in range(_NCHUNK):
        if k + 1 < _NCHUNK:
            nxt = fire(k + 1, bufs[(k + 1) % 2], sems[(k + 1) % 2])
        for cp in pending:
            cp.wait()
        accumulate(bufs[k % 2])
        pltpu.sync_copy(out_v, out_hbm.at[pl.ds(wid * _BPW + k * _CHUNK_B,
                                                _CHUNK_B)])
        if k + 1 < _NCHUNK:
            pending = nxt


@jax.jit
def _sc_embed(c3, offs, tbl):
    mesh = plsc.VectorSubcoreMesh(core_axis_name="c", subcore_axis_name="s",
                                  num_cores=_NC, num_subcores=_NS)
    return pl.kernel(
        _body,
        out_type=jax.ShapeDtypeStruct((_BATCH, _EMBED), jnp.float32),
        mesh=mesh,
        scratch_types=[
            pltpu.VMEM((_NGROUP, _GROUP), jnp.int32),
            pltpu.VMEM((_PERIOD, _GROUP), jnp.int32),
            pltpu.VMEM((_CHUNK_ROWS, _L), jnp.int32),
            pltpu.VMEM((_CHUNK_ROWS, _L), jnp.int32),
            pltpu.VMEM((_CHUNK_B, _EMBED), jnp.float32),
            pltpu.SemaphoreType.DMA,
            pltpu.SemaphoreType.DMA,
        ],
        compiler_params=pltpu.CompilerParams(use_tc_tiling_on_sc=False),
    )(c3, offs, tbl)


def kernel(c, tables):
    c3 = c.reshape(_NW, _NGROUP, _GROUP)
    t2 = tables.transpose(0, 2, 1)
    staged = _repack(t2)
    tbl = staged.reshape(_N_FIELDS * _VP, _L)
    offs = jnp.asarray(
        (np.arange(_PERIOD * _GROUP).reshape(_PERIOD, _GROUP) % _N_FIELDS)
        * _VP, dtype=np.int32)
    return _sc_embed(c3, offs, tbl)


# speedup vs baseline: 22.9291x; 2.0462x over previous
import jax
import jax.numpy as jnp
import numpy as np
from jax import lax
from jax.experimental import pallas as pl
from jax.experimental.pallas import tpu as pltpu
from jax.experimental.pallas import tpu_sc as plsc

_N_FIELDS = 26
_VOCAB_P1 = 100001
_VP2 = 100352
_EMBED = 32
_BATCH = 16384

_NC, _NS, _L = 2, 16, 16
_NW = _NC * _NS
_BPW = _BATCH // _NW
_CHUNK_B = 64
_NCHUNK = _BPW // _CHUNK_B
_NPLANE = 16
_NLIST = _N_FIELDS * _NPLANE
_CELEM = _NLIST * _CHUNK_B

_BLKV2 = 14336
_NVB2 = _VP2 // _BLKV2
_VB128 = _BLKV2 // 128


def _pack_body(x_ref, o_ref):
    x = x_ref[0]
    lo = lax.bitcast_convert_type(
        x[0:16, :].astype(jnp.bfloat16), jnp.uint16).astype(jnp.uint32)
    hi = lax.bitcast_convert_type(
        x[16:32, :].astype(jnp.bfloat16), jnp.uint16).astype(jnp.uint32)
    packed = lax.bitcast_convert_type(lo | (hi << 16), jnp.int32)
    o_ref[:, :, :] = packed.reshape(_NPLANE, _VB128, 128)


@jax.jit
def _pack(t2):
    return pl.pallas_call(
        _pack_body,
        grid=(_N_FIELDS, _NVB2),
        in_specs=[pl.BlockSpec((1, _EMBED, _BLKV2), lambda f, v: (f, 0, v))],
        out_specs=pl.BlockSpec((_NPLANE, _VB128, 128),
                               lambda f, v: (f, v, 0)),
        out_shape=jax.ShapeDtypeStruct((_N_FIELDS * _NPLANE, _VP2 // 128, 128),
                                       jnp.int32),
    )(t2)


def _body(c_hbm, pofs_hbm, tbl_hbm, tbl2_hbm, out_hbm,
          idx_v, pofs_v, lists0, lists1, data0, data1, out_v, sem0, sem1):
    wid = lax.axis_index("s") * _NC + lax.axis_index("c")

    pltpu.sync_copy(c_hbm.at[wid], idx_v)
    pltpu.sync_copy(pofs_hbm, pofs_v)

    inv = jnp.float32(1.0 / _N_FIELDS)
    msk = jnp.int32(-65536)

    def build_and_fire(k, lists, data, sem):
        @pl.loop(0, _N_FIELDS)
        def _build(f):
            for v4 in range(_CHUNK_B // _L):
                base = idx_v[f, pl.ds(k * _CHUNK_B + v4 * _L, _L)]
                for j in range(_NPLANE):
                    li = f * _NPLANE + j
                    lists[li, pl.ds(v4 * _L, _L)] = base + pofs_v[li, :]

        @pl.loop(0, _NLIST)
        def _fire(g):
            pltpu.async_copy(tbl_hbm.at[lists.at[g]], data.at[g], sem)

    def drain(data, sem):
        pltpu.make_async_copy(tbl2_hbm, data, sem).wait()

    def accumulate(data):
        @pl.loop(0, _NPLANE)
        def _acc(j):
            for b4 in range(_CHUNK_B // _L):
                sl = pl.ds(b4 * _L, _L)
                v = data[j, sl]
                lo = lax.bitcast_convert_type(v << 16, jnp.float32)
                hi = lax.bitcast_convert_type(v & msk, jnp.float32)
                for f in range(1, _N_FIELDS):
                    v = data[f * _NPLANE + j, sl]
                    lo = lo + lax.bitcast_convert_type(v << 16, jnp.float32)
                    hi = hi + lax.bitcast_convert_type(v & msk, jnp.float32)
                out_v[j, sl] = lo * inv
                out_v[j + _NPLANE, sl] = hi * inv

    lists = (lists0, lists1)
    datas = (data0, data1)
    sems = (sem0, sem1)
    build_and_fire(0, lists[0], datas[0], sems[0])
    for k in range(_NCHUNK):
        if k + 1 < _NCHUNK:
            build_and_fire(k + 1, lists[(k + 1) % 2], datas[(k + 1) % 2],
                           sems[(k + 1) % 2])
        drain(datas[k % 2], sems[k % 2])
        accumulate(datas[k % 2])
        pltpu.sync_copy(out_v,
                        out_hbm.at[:, pl.ds(wid * _BPW + k * _CHUNK_B,
                                            _CHUNK_B)])


@jax.jit
def _sc_embed(c3, pofs, tbl, tbl2):
    mesh = plsc.VectorSubcoreMesh(core_axis_name="c", subcore_axis_name="s",
                                  num_cores=_NC, num_subcores=_NS)
    return pl.kernel(
        _body,
        out_type=jax.ShapeDtypeStruct((_EMBED, _BATCH), jnp.float32),
        mesh=mesh,
        scratch_types=[
            pltpu.VMEM((_N_FIELDS, _BPW), jnp.int32),
            pltpu.VMEM((_NLIST, _L), jnp.int32),
            pltpu.VMEM((_NLIST, _CHUNK_B), jnp.int32),
            pltpu.VMEM((_NLIST, _CHUNK_B), jnp.int32),
            pltpu.VMEM((_NLIST, _CHUNK_B), jnp.int32),
            pltpu.VMEM((_NLIST, _CHUNK_B), jnp.int32),
            pltpu.VMEM((_EMBED, _CHUNK_B), jnp.float32),
            pltpu.SemaphoreType.DMA,
            pltpu.SemaphoreType.DMA,
        ],
        compiler_params=pltpu.CompilerParams(use_tc_tiling_on_sc=False),
    )(c3, pofs, tbl, tbl2)


def kernel(c, tables):
    t2 = tables.transpose(0, 2, 1)
    staged = _pack(t2)
    tbl = staged.reshape(_NLIST * _VP2)
    tbl2 = jnp.zeros((_NLIST, _CHUNK_B), jnp.int32)
    c3 = c.T.reshape(_N_FIELDS, _NW, _BPW).transpose(1, 0, 2)
    li = np.arange(_NLIST, dtype=np.int64)
    pofs = jnp.asarray(np.broadcast_to((li * _VP2)[:, None], (_NLIST, _L))
                       .astype(np.int32))
    out_t = _sc_embed(c3, pofs, tbl, tbl2)
    return out_t.T


# speedup vs baseline: 26.3004x; 1.1470x over previous
import jax
import jax.numpy as jnp
import numpy as np
from jax import lax
from jax.experimental import pallas as pl
from jax.experimental.pallas import tpu as pltpu
from jax.experimental.pallas import tpu_sc as plsc

_N_FIELDS = 26
_NFG = 13
_VOCAB_P1 = 100001
_VP2 = 100352
_EMBED = 32
_BATCH = 16384

_NC, _NS, _L = 2, 16, 16
_NW = _NC * _NS
_BPW = _BATCH // _NW
_CHUNK_B = 64
_NCHUNK = _BPW // _CHUNK_B
_NPLANE = 16
_NLIST = _NFG * _NPLANE
_CELEM = _NLIST * _CHUNK_B

_BLKV2 = 14336
_NVB2 = _VP2 // _BLKV2
_VB128 = _BLKV2 // 128


def _pack_body(x_ref, o_ref):
    x = x_ref[0]
    lo = lax.bitcast_convert_type(
        x[0:16, :].astype(jnp.bfloat16), jnp.uint16).astype(jnp.uint32)
    hi = lax.bitcast_convert_type(
        x[16:32, :].astype(jnp.bfloat16), jnp.uint16).astype(jnp.uint32)
    packed = lax.bitcast_convert_type(lo | (hi << 16), jnp.int32)
    o_ref[:, :, :] = packed.reshape(_NPLANE, _VB128, 128)


import functools


@functools.partial(jax.jit, static_argnums=1)
def _pack(t2, g):
    return pl.pallas_call(
        _pack_body,
        grid=(_NFG, _NVB2),
        in_specs=[pl.BlockSpec((1, _EMBED, _BLKV2),
                               lambda f, v, g=g: (g * _NFG + f, 0, v))],
        out_specs=pl.BlockSpec((_NPLANE, _VB128, 128),
                               lambda f, v: (f, v, 0)),
        out_shape=jax.ShapeDtypeStruct((_NFG * _NPLANE, _VP2 // 128, 128),
                                       jnp.int32),
    )(t2)


def _body(c_hbm, pofs_hbm, tbl_hbm, tbl2_hbm, out_hbm,
          idx_v, pofs_v, lists0, lists1, data0, data1, out_v, sem0, sem1):
    wid = lax.axis_index("s") * _NC + lax.axis_index("c")

    pltpu.sync_copy(c_hbm.at[wid], idx_v)
    pltpu.sync_copy(pofs_hbm, pofs_v)

    inv = jnp.float32(1.0 / _N_FIELDS)
    msk = jnp.int32(-65536)

    def build_and_fire(k, lists, data, sem):
        @pl.loop(0, _NFG)
        def _build(f):
            for v4 in range(_CHUNK_B // _L):
                base = idx_v[f, pl.ds(k * _CHUNK_B + v4 * _L, _L)]
                for j in range(_NPLANE):
                    li = f * _NPLANE + j
                    lists[li, pl.ds(v4 * _L, _L)] = base + pofs_v[li, :]

        @pl.loop(0, _NLIST)
        def _fire(g):
            pltpu.async_copy(tbl_hbm.at[lists.at[g]], data.at[g], sem)

    def drain(data, sem):
        pltpu.make_async_copy(tbl2_hbm, data, sem).wait()

    def accumulate(data):
        @pl.loop(0, _NPLANE)
        def _acc(j):
            for b4 in range(_CHUNK_B // _L):
                sl = pl.ds(b4 * _L, _L)
                v = data[j, sl]
                lo = lax.bitcast_convert_type(v << 16, jnp.float32)
                hi = lax.bitcast_convert_type(v & msk, jnp.float32)
                for f in range(1, _NFG):
                    v = data[f * _NPLANE + j, sl]
                    lo = lo + lax.bitcast_convert_type(v << 16, jnp.float32)
                    hi = hi + lax.bitcast_convert_type(v & msk, jnp.float32)
                out_v[j, sl] = lo * inv
                out_v[j + _NPLANE, sl] = hi * inv

    lists = (lists0, lists1)
    datas = (data0, data1)
    sems = (sem0, sem1)
    build_and_fire(0, lists[0], datas[0], sems[0])
    for k in range(_NCHUNK):
        if k + 1 < _NCHUNK:
            build_and_fire(k + 1, lists[(k + 1) % 2], datas[(k + 1) % 2],
                           sems[(k + 1) % 2])
        drain(datas[k % 2], sems[k % 2])
        accumulate(datas[k % 2])
        pltpu.sync_copy(out_v,
                        out_hbm.at[:, pl.ds(wid * _BPW + k * _CHUNK_B,
                                            _CHUNK_B)])


@jax.jit
def _sc_embed(c3, pofs, tbl, tbl2):
    mesh = plsc.VectorSubcoreMesh(core_axis_name="c", subcore_axis_name="s",
                                  num_cores=_NC, num_subcores=_NS)
    return pl.kernel(
        _body,
        out_type=jax.ShapeDtypeStruct((_EMBED, _BATCH), jnp.float32),
        mesh=mesh,
        scratch_types=[
            pltpu.VMEM((_NFG, _BPW), jnp.int32),
            pltpu.VMEM((_NLIST, _L), jnp.int32),
            pltpu.VMEM((_NLIST, _CHUNK_B), jnp.int32),
            pltpu.VMEM((_NLIST, _CHUNK_B), jnp.int32),
            pltpu.VMEM((_NLIST, _CHUNK_B), jnp.int32),
            pltpu.VMEM((_NLIST, _CHUNK_B), jnp.int32),
            pltpu.VMEM((_EMBED, _CHUNK_B), jnp.float32),
            pltpu.SemaphoreType.DMA,
            pltpu.SemaphoreType.DMA,
        ],
        compiler_params=pltpu.CompilerParams(use_tc_tiling_on_sc=False),
    )(c3, pofs, tbl, tbl2)


def kernel(c, tables):
    t2 = tables.transpose(0, 2, 1)
    cT = c.T.reshape(_N_FIELDS, _NW, _BPW)
    tbl2 = jnp.zeros((_NLIST, _CHUNK_B), jnp.int32)
    li = np.arange(_NLIST, dtype=np.int64)
    pofs = jnp.asarray(np.broadcast_to((li * _VP2)[:, None], (_NLIST, _L))
                       .astype(np.int32))
    parts = []
    for g in range(2):
        staged = _pack(t2, g)
        tbl = staged.reshape(_NLIST * _VP2)
        c3 = cT[g * _NFG:(g + 1) * _NFG].transpose(1, 0, 2)
        parts.append(_sc_embed(c3, pofs, tbl, tbl2))
    return (parts[0] + parts[1]).T
